# Initial kernel scaffold; baseline (speedup 1.0000x reference)
import jax
import jax.numpy as jnp
from jax.experimental import pallas as pl


def kernel(x, edge_index, W1, a_src1, a_dst1, b1, W2, a_src2, a_dst2, b2, Wv, a_srcv, a_dstv, bv, lin_W, lin_b):
    raise NotImplementedError("write your pallas kernel here")


# speedup vs baseline: 24.4848x; 24.4848x over previous
import functools

import jax
import jax.numpy as jnp
from jax import lax
from jax.experimental import pallas as pl
from jax.experimental.pallas import tpu as pltpu
from jax.experimental.pallas import tpu_sc as plsc

N = 10000
NP = 10240
D = 128
E = 320000
CHUNK = 128
NCH = E // CHUNK
NC = 2
NT = 16
NW = NC * NT
RPT = NP // NT
BR = 1024
GRID = NP // BR
EPS = 1e-16
NEG = -3.0e38


def _t1_body(x_ref, w_ref, avs_ref, avd_ref, h_ref, sa_ref, sd_ref):
    h = jnp.dot(x_ref[...], w_ref[...], preferred_element_type=jnp.float32)
    h_ref[...] = h
    sa_ref[...] = jnp.sum(h * avs_ref[...][None, :], axis=1).reshape(8, 128)
    sd_ref[...] = jnp.sum(h * avd_ref[...][None, :], axis=1).reshape(8, 128)


def _t1(xp, W1, a_src1, a_dst1):
    return pl.pallas_call(
        _t1_body,
        grid=(GRID,),
        in_specs=[
            pl.BlockSpec((BR, D), lambda i: (i, 0)),
            pl.BlockSpec((D, D), lambda i: (0, 0)),
            pl.BlockSpec((D,), lambda i: (0,)),
            pl.BlockSpec((D,), lambda i: (0,)),
        ],
        out_specs=[
            pl.BlockSpec((BR, D), lambda i: (i, 0)),
            pl.BlockSpec((8, 128), lambda i: (i, 0)),
            pl.BlockSpec((8, 128), lambda i: (i, 0)),
        ],
        out_shape=[
            jax.ShapeDtypeStruct((NP, D), jnp.float32),
            jax.ShapeDtypeStruct((NP // 128, 128), jnp.float32),
            jax.ShapeDtypeStruct((NP // 128, 128), jnp.float32),
        ],
    )(xp, W1, a_src1, a_dst1)


def _t2_body(p_ref, s_ref, b1_ref, w2_ref, wv_ref, a2s_ref, a2d_ref,
             avs_ref, avd_ref,
             h2_ref, hv_ref, t2s_ref, t2d_ref, tvs_ref, tvd_ref):
    ssum = jnp.sum(s_ref[...], axis=0)
    p = p_ref[0] + p_ref[1]
    v = p * (1.0 / (ssum + EPS))[:, None] + b1_ref[...][None, :]
    h1 = jnp.where(v > 0, v, jnp.exp(v) - 1.0)
    h2 = jnp.dot(h1, w2_ref[...], preferred_element_type=jnp.float32)
    hv = jnp.dot(h1, wv_ref[...], preferred_element_type=jnp.float32)
    h2_ref[...] = h2
    hv_ref[...] = hv
    t2s_ref[...] = jnp.sum(h2 * a2s_ref[...][None, :], axis=1).reshape(8, 128)
    t2d_ref[...] = jnp.sum(h2 * a2d_ref[...][None, :], axis=1).reshape(8, 128)
    tvs_ref[...] = jnp.sum(hv * avs_ref[...][None, :], axis=1).reshape(8, 128)
    tvd_ref[...] = jnp.sum(hv * avd_ref[...][None, :], axis=1).reshape(8, 128)


def _t2(out1p, s1p, b1, W2, Wv, a_src2, a_dst2, a_srcv, a_dstv):
    vec = pl.BlockSpec((D,), lambda i: (0,))
    mat = pl.BlockSpec((D, D), lambda i: (0, 0))
    tab = pl.BlockSpec((8, 128), lambda i: (i, 0))
    return pl.pallas_call(
        _t2_body,
        grid=(GRID,),
        in_specs=[
            pl.BlockSpec((NC, BR, D), lambda i: (0, i, 0)),
            pl.BlockSpec((NC, BR), lambda i: (0, i)),
            vec, mat, mat, vec, vec, vec, vec,
        ],
        out_specs=[
            pl.BlockSpec((BR, D), lambda i: (i, 0)),
            pl.BlockSpec((BR, D), lambda i: (i, 0)),
            tab, tab, tab, tab,
        ],
        out_shape=[
            jax.ShapeDtypeStruct((NP, D), jnp.float32),
            jax.ShapeDtypeStruct((NP, D), jnp.float32),
            jax.ShapeDtypeStruct((NP // 128, 128), jnp.float32),
            jax.ShapeDtypeStruct((NP // 128, 128), jnp.float32),
            jax.ShapeDtypeStruct((NP // 128, 128), jnp.float32),
            jax.ShapeDtypeStruct((NP // 128, 128), jnp.float32),
        ],
    )(out1p, s1p, b1, W2, Wv, a_src2, a_dst2, a_srcv, a_dstv)


def _t3_body(p2_ref, s2_ref, b2_ref, pv_ref, sv_ref, bv_ref,
             tvs_ref, tvd_ref, hv_ref, cv_ref, lw_ref, lb_ref,
             mean_ref, var_ref):
    s2 = jnp.sum(s2_ref[...], axis=0)
    m = (p2_ref[0] + p2_ref[1]) * (1.0 / (s2 + EPS))[:, None] + b2_ref[...][None, :]
    m = jnp.maximum(m, 0.0)
    mean_ref[...] = (
        jnp.dot(m, lw_ref[...], preferred_element_type=jnp.float32)
        + lb_ref[...][None, :]
    )
    cv = cv_ref[0]
    es = tvs_ref[...].reshape(BR) + tvd_ref[...].reshape(BR)
    es = jnp.maximum(es, 0.2 * es)
    exs = jnp.exp(es - cv)
    sv = jnp.sum(sv_ref[...], axis=0) + exs
    num = pv_ref[0] + pv_ref[1] + exs[:, None] * hv_ref[...]
    var_ref[...] = num * (1.0 / (sv + EPS))[:, None] + bv_ref[...][None, :]


def _t3(out2p, s2p, b2, outvp, svp, bv, tvs, tvd, hv, cv128, lin_W, lin_b):
    vec = pl.BlockSpec((D,), lambda i: (0,))
    mat = pl.BlockSpec((D, D), lambda i: (0, 0))
    tab = pl.BlockSpec((8, 128), lambda i: (i, 0))
    return pl.pallas_call(
        _t3_body,
        grid=(GRID,),
        in_specs=[
            pl.BlockSpec((NC, BR, D), lambda i: (0, i, 0)),
            pl.BlockSpec((NC, BR), lambda i: (0, i)),
            vec,
            pl.BlockSpec((NC, BR, D), lambda i: (0, i, 0)),
            pl.BlockSpec((NC, BR), lambda i: (0, i)),
            vec,
            tab, tab,
            pl.BlockSpec((BR, D), lambda i: (i, 0)),
            vec, mat, vec,
        ],
        out_specs=[
            pl.BlockSpec((BR, D), lambda i: (i, 0)),
            pl.BlockSpec((BR, D), lambda i: (i, 0)),
        ],
        out_shape=[
            jax.ShapeDtypeStruct((NP, D), jnp.float32),
            jax.ShapeDtypeStruct((NP, D), jnp.float32),
        ],
    )(out2p, s2p, b2, outvp, svp, bv, tvs, tvd, hv, cv128, lin_W, lin_b)


def _make_sc(mode):
    mesh = plsc.VectorSubcoreMesh(core_axis_name="c", subcore_axis_name="s")

    out_type = [
        jax.ShapeDtypeStruct((NC, NP, D), jnp.float32),
        jax.ShapeDtypeStruct((NC, NP), jnp.float32),
    ]
    if mode == 1:
        out_type.append(jax.ShapeDtypeStruct((NCH, CHUNK), jnp.float32))
    if mode == 2:
        out_type.append(jax.ShapeDtypeStruct((NCH, CHUNK), jnp.float32))
        out_type.append(jax.ShapeDtypeStruct((16,), jnp.float32))

    scratch = [
        pltpu.VMEM((NP,), jnp.float32),
        pltpu.VMEM((NP,), jnp.float32),
        pltpu.VMEM((CHUNK, D), jnp.float32),
        pltpu.VMEM((1, CHUNK), jnp.int32),
        pltpu.VMEM((1, CHUNK), jnp.int32),
        pltpu.VMEM((CHUNK,), jnp.float32),
        pltpu.VMEM_SHARED((NP, D), jnp.float32),
        pltpu.VMEM_SHARED((NP,), jnp.float32),
        pltpu.SemaphoreType.DMA,
    ]
    if mode == 2:
        scratch += [
            pltpu.VMEM((CHUNK,), jnp.float32),
            pltpu.VMEM((16,), jnp.float32),
        ]

    def body(*refs):
        it = iter(refs)
        h_hbm = next(it); asv_hbm = next(it); adv_hbm = next(it)
        src_hbm = next(it); dst_hbm = next(it)
        if mode == 2:
            ex2_hbm = next(it); s2p_hbm = next(it)
        outp_hbm = next(it); sp_hbm = next(it)
        if mode == 1:
            exh_hbm = next(it)
        if mode == 2:
            alh_hbm = next(it); cv_hbm = next(it)
        as_t = next(it); ad_t = next(it); rows_v = next(it)
        srcix = next(it); dstix = next(it); exf = next(it)
        out_sh = next(it); s_sh = next(it); sem = next(it)
        if mode == 2:
            alf = next(it); cvb = next(it)

        cid = lax.axis_index("c")
        sid = lax.axis_index("s")
        wid = sid * NC + cid

        pltpu.sync_copy(asv_hbm, as_t)
        pltpu.sync_copy(adv_hbm, ad_t)

        zero16 = jnp.zeros((16,), jnp.float32)

        def mx_body(i, carry):
            ma, md = carry
            return (jnp.maximum(ma, as_t[pl.ds(i * 16, 16)]),
                    jnp.maximum(md, ad_t[pl.ds(i * 16, 16)]))
        ma, md = lax.fori_loop(0, NP // 16, mx_body,
                               (jnp.full((16,), NEG, jnp.float32),
                                jnp.full((16,), NEG, jnp.float32)))

        lanes = lax.iota(jnp.int32, 16)

        def allmax(v):
            for sh in (1, 2, 4, 8):
                exf[pl.ds(0, 16)] = v
                v = jnp.maximum(v, plsc.load_gather(exf, [lanes ^ sh]))
            return v

        C = allmax(ma) + allmax(md)

        def zrow_body(r, _):
            for t in range(D // 16):
                rows_v[r, pl.ds(t * 16, 16)] = zero16
            return 0
        lax.fori_loop(0, CHUNK, zrow_body, 0)

        def zex_body(i, _):
            exf[pl.ds(i * 16, 16)] = zero16
            return 0
        lax.fori_loop(0, CHUNK // 16, zex_body, 0)

        base = sid * RPT
        for k in range(RPT // CHUNK):
            pltpu.sync_copy(rows_v, out_sh.at[pl.ds(base + k * CHUNK, CHUNK)])

        @pl.when(sid == 0)
        def _():
            for k in range(NP // CHUNK):
                pltpu.sync_copy(exf, s_sh.at[pl.ds(k * CHUNK, CHUNK)])

        plsc.subcore_barrier()

        nch_w = NCH // NW + jnp.where(wid < NCH % NW, 1, 0)

        def chunk_body(k, _):
            ci = wid + k * NW
            pltpu.sync_copy(src_hbm.at[ci], srcix)
            pltpu.sync_copy(dst_hbm.at[ci], dstix)

            desc = pltpu.async_copy(h_hbm.at[srcix.at[0]], rows_v, sem)

            for t in range(CHUNK // 16):
                sl = pl.ds(t * 16, 16)
                srcv = srcix[0, sl]
                dstv = dstix[0, sl]
                av = plsc.load_gather(as_t, [srcv])
                dv = plsc.load_gather(ad_t, [dstv])
                e = av + dv
                e = jnp.maximum(e, 0.2 * e)
                exf[sl] = jnp.exp(e - C)

            if mode == 1:
                pltpu.sync_copy(exf, exh_hbm.at[ci])

            pltpu.sync_copy(exf, s_sh.at[dstix.at[0]], add=True)

            desc.wait()

            def scale_body(r, _):
                bc = plsc.load_gather(
                    exf, [jnp.broadcast_to(r, (16,)).astype(jnp.int32)])
                for t in range(D // 16):
                    sl = pl.ds(t * 16, 16)
                    rows_v[r, sl] = rows_v[r, sl] * bc
                return 0
            lax.fori_loop(0, CHUNK, scale_body, 0)

            pltpu.sync_copy(rows_v, out_sh.at[dstix.at[0]], add=True)
            return 0

        lax.fori_loop(0, nch_w, chunk_body, 0)

        if mode == 2:
            pltpu.sync_copy(s2p_hbm.at[0], as_t)
            pltpu.sync_copy(s2p_hbm.at[1], ad_t)

            def alpha_body(k, _):
                ci = wid + k * NW
                pltpu.sync_copy(dst_hbm.at[ci], dstix)
                pltpu.sync_copy(ex2_hbm.at[ci], exf)
                for t in range(CHUNK // 16):
                    sl = pl.ds(t * 16, 16)
                    dstv = dstix[0, sl]
                    s2g = (plsc.load_gather(as_t, [dstv])
                           + plsc.load_gather(ad_t, [dstv]))
                    alf[sl] = exf[sl] / (s2g + EPS)
                pltpu.sync_copy(alf, alh_hbm.at[ci])
                return 0
            lax.fori_loop(0, nch_w, alpha_body, 0)

            @pl.when(wid == 0)
            def _():
                cvb[pl.ds(0, 16)] = C
                pltpu.sync_copy(cvb, cv_hbm)

        plsc.subcore_barrier()

        pltpu.sync_copy(out_sh.at[pl.ds(base, RPT)],
                        outp_hbm.at[cid, pl.ds(base, RPT)])
        pltpu.sync_copy(s_sh.at[pl.ds(base, RPT)],
                        sp_hbm.at[cid, pl.ds(base, RPT)])

    return pl.kernel(body, out_type=out_type, mesh=mesh,
                     scratch_types=scratch,
                     compiler_params=pltpu.CompilerParams(
                         needs_layout_passes=False))


_sc1 = _make_sc(0)
_sc2 = _make_sc(1)
_scv = _make_sc(2)


def kernel(x, edge_index, W1, a_src1, a_dst1, b1, W2, a_src2, a_dst2, b2,
           Wv, a_srcv, a_dstv, bv, lin_W, lin_b):
    xp = jnp.zeros((NP, D), jnp.float32).at[:N].set(x)
    src_r = edge_index[0].reshape(NCH, 1, CHUNK)
    dst_r = edge_index[1].reshape(NCH, 1, CHUNK)

    h1m, sa1, sd1 = _t1(xp, W1, a_src1, a_dst1)
    out1p, s1p = _sc1(h1m, sa1.reshape(NP), sd1.reshape(NP), src_r, dst_r)

    h2, hv, t2s, t2d, tvs, tvd = _t2(out1p, s1p, b1, W2, Wv,
                                     a_src2, a_dst2, a_srcv, a_dstv)
    out2p, s2p, exh = _sc2(h2, t2s.reshape(NP), t2d.reshape(NP), src_r, dst_r)
    outvp, svp, alh, cv16 = _scv(hv, tvs.reshape(NP), tvd.reshape(NP),
                                 src_r, dst_r, exh, s2p)

    mean, var = _t3(out2p, s2p, b2, outvp, svp, bv, tvs, tvd, hv,
                    jnp.tile(cv16, 8), lin_W, lin_b)
    return (mean[:N], var[:N], edge_index, alh.reshape(E))


# speedup vs baseline: 28.8325x; 1.1776x over previous
import functools

import jax
import jax.numpy as jnp
from jax import lax
from jax.experimental import pallas as pl
from jax.experimental.pallas import tpu as pltpu
from jax.experimental.pallas import tpu_sc as plsc

N = 10000
NP = 10240
D = 128
E = 320000
CHUNK = 128
HALF = CHUNK // 2
NCH = E // CHUNK
NC = 2
NT = 16
NW = NC * NT
RPT = NP // NT
BR = 1024
GRID = NP // BR
EPS = 1e-16
NEG = -3.0e38


def _t1_body(x_ref, w_ref, avs_ref, avd_ref, h_ref, sa_ref, sd_ref):
    h = jnp.dot(x_ref[...], w_ref[...], preferred_element_type=jnp.float32)
    h_ref[...] = h
    sa_ref[...] = jnp.sum(h * avs_ref[...][None, :], axis=1).reshape(8, 128)
    sd_ref[...] = jnp.sum(h * avd_ref[...][None, :], axis=1).reshape(8, 128)


def _t1(xp, W1, a_src1, a_dst1):
    return pl.pallas_call(
        _t1_body,
        grid=(GRID,),
        in_specs=[
            pl.BlockSpec((BR, D), lambda i: (i, 0)),
            pl.BlockSpec((D, D), lambda i: (0, 0)),
            pl.BlockSpec((D,), lambda i: (0,)),
            pl.BlockSpec((D,), lambda i: (0,)),
        ],
        out_specs=[
            pl.BlockSpec((BR, D), lambda i: (i, 0)),
            pl.BlockSpec((8, 128), lambda i: (i, 0)),
            pl.BlockSpec((8, 128), lambda i: (i, 0)),
        ],
        out_shape=[
            jax.ShapeDtypeStruct((NP, D), jnp.float32),
            jax.ShapeDtypeStruct((NP // 128, 128), jnp.float32),
            jax.ShapeDtypeStruct((NP // 128, 128), jnp.float32),
        ],
    )(xp, W1, a_src1, a_dst1)


def _t2_body(p_ref, s_ref, b1_ref, w2_ref, wv_ref, a2s_ref, a2d_ref,
             avs_ref, avd_ref,
             h2_ref, hv_ref, t2s_ref, t2d_ref, tvs_ref, tvd_ref):
    ssum = jnp.sum(s_ref[...], axis=0)
    p = p_ref[0] + p_ref[1]
    v = p * (1.0 / (ssum + EPS))[:, None] + b1_ref[...][None, :]
    h1 = jnp.where(v > 0, v, jnp.exp(v) - 1.0)
    h2 = jnp.dot(h1, w2_ref[...], preferred_element_type=jnp.float32)
    hv = jnp.dot(h1, wv_ref[...], preferred_element_type=jnp.float32)
    h2_ref[...] = h2
    hv_ref[...] = hv
    t2s_ref[...] = jnp.sum(h2 * a2s_ref[...][None, :], axis=1).reshape(8, 128)
    t2d_ref[...] = jnp.sum(h2 * a2d_ref[...][None, :], axis=1).reshape(8, 128)
    tvs_ref[...] = jnp.sum(hv * avs_ref[...][None, :], axis=1).reshape(8, 128)
    tvd_ref[...] = jnp.sum(hv * avd_ref[...][None, :], axis=1).reshape(8, 128)


def _t2(out1p, s1p, b1, W2, Wv, a_src2, a_dst2, a_srcv, a_dstv):
    vec = pl.BlockSpec((D,), lambda i: (0,))
    mat = pl.BlockSpec((D, D), lambda i: (0, 0))
    tab = pl.BlockSpec((8, 128), lambda i: (i, 0))
    return pl.pallas_call(
        _t2_body,
        grid=(GRID,),
        in_specs=[
            pl.BlockSpec((NC, BR, D), lambda i: (0, i, 0)),
            pl.BlockSpec((NC, BR), lambda i: (0, i)),
            vec, mat, mat, vec, vec, vec, vec,
        ],
        out_specs=[
            pl.BlockSpec((BR, D), lambda i: (i, 0)),
            pl.BlockSpec((BR, D), lambda i: (i, 0)),
            tab, tab, tab, tab,
        ],
        out_shape=[
            jax.ShapeDtypeStruct((NP, D), jnp.float32),
            jax.ShapeDtypeStruct((NP, D), jnp.float32),
            jax.ShapeDtypeStruct((NP // 128, 128), jnp.float32),
            jax.ShapeDtypeStruct((NP // 128, 128), jnp.float32),
            jax.ShapeDtypeStruct((NP // 128, 128), jnp.float32),
            jax.ShapeDtypeStruct((NP // 128, 128), jnp.float32),
        ],
    )(out1p, s1p, b1, W2, Wv, a_src2, a_dst2, a_srcv, a_dstv)


def _t2b_body(s_ref, o_ref):
    o_ref[...] = jnp.sum(s_ref[...], axis=0).reshape(8, 128)


def _t2b(s2p):
    return pl.pallas_call(
        _t2b_body,
        grid=(GRID,),
        in_specs=[pl.BlockSpec((NC, BR), lambda i: (0, i))],
        out_specs=pl.BlockSpec((8, 128), lambda i: (i, 0)),
        out_shape=jax.ShapeDtypeStruct((NP // 128, 128), jnp.float32),
    )(s2p)


def _t3_body(p2_ref, s2_ref, b2_ref, pv_ref, sv_ref, bv_ref,
             tvs_ref, tvd_ref, hv_ref, cv_ref, lw_ref, lb_ref,
             mean_ref, var_ref):
    s2 = jnp.sum(s2_ref[...], axis=0)
    m = (p2_ref[0] + p2_ref[1]) * (1.0 / (s2 + EPS))[:, None] + b2_ref[...][None, :]
    m = jnp.maximum(m, 0.0)
    mean_ref[...] = (
        jnp.dot(m, lw_ref[...], preferred_element_type=jnp.float32)
        + lb_ref[...][None, :]
    )
    cv = cv_ref[0]
    es = tvs_ref[...].reshape(BR) + tvd_ref[...].reshape(BR)
    es = jnp.maximum(es, 0.2 * es)
    exs = jnp.exp(es - cv)
    sv = jnp.sum(sv_ref[...], axis=0) + exs
    num = pv_ref[0] + pv_ref[1] + exs[:, None] * hv_ref[...]
    var_ref[...] = num * (1.0 / (sv + EPS))[:, None] + bv_ref[...][None, :]


def _t3(out2p, s2p, b2, outvp, svp, bv, tvs, tvd, hv, cv128, lin_W, lin_b):
    vec = pl.BlockSpec((D,), lambda i: (0,))
    mat = pl.BlockSpec((D, D), lambda i: (0, 0))
    tab = pl.BlockSpec((8, 128), lambda i: (i, 0))
    return pl.pallas_call(
        _t3_body,
        grid=(GRID,),
        in_specs=[
            pl.BlockSpec((NC, BR, D), lambda i: (0, i, 0)),
            pl.BlockSpec((NC, BR), lambda i: (0, i)),
            vec,
            pl.BlockSpec((NC, BR, D), lambda i: (0, i, 0)),
            pl.BlockSpec((NC, BR), lambda i: (0, i)),
            vec,
            tab, tab,
            pl.BlockSpec((BR, D), lambda i: (i, 0)),
            vec, mat, vec,
        ],
        out_specs=[
            pl.BlockSpec((BR, D), lambda i: (i, 0)),
            pl.BlockSpec((BR, D), lambda i: (i, 0)),
        ],
        out_shape=[
            jax.ShapeDtypeStruct((NP, D), jnp.float32),
            jax.ShapeDtypeStruct((NP, D), jnp.float32),
        ],
    )(out2p, s2p, b2, outvp, svp, bv, tvs, tvd, hv, cv128, lin_W, lin_b)


def _make_sc(mode):
    mesh = plsc.VectorSubcoreMesh(core_axis_name="c", subcore_axis_name="s")

    out_type = [
        jax.ShapeDtypeStruct((NC, NP, D), jnp.float32),
        jax.ShapeDtypeStruct((NC, NP), jnp.float32),
    ]
    if mode == 1:
        out_type.append(jax.ShapeDtypeStruct((NCH, CHUNK), jnp.float32))
    if mode == 2:
        out_type.append(jax.ShapeDtypeStruct((NCH, CHUNK), jnp.float32))
        out_type.append(jax.ShapeDtypeStruct((16,), jnp.float32))

    scratch = [
        pltpu.VMEM((NP,), jnp.float32),
        pltpu.VMEM((NP,), jnp.float32),
        pltpu.VMEM((CHUNK, D), jnp.float32),
        pltpu.VMEM((2, HALF), jnp.int32),
        pltpu.VMEM((2, HALF), jnp.int32),
        pltpu.VMEM((CHUNK,), jnp.float32),
        pltpu.VMEM_SHARED((NP, D), jnp.float32),
        pltpu.VMEM_SHARED((NP,), jnp.float32),
        pltpu.SemaphoreType.DMA,
        pltpu.SemaphoreType.DMA,
        pltpu.SemaphoreType.DMA,
        pltpu.SemaphoreType.DMA,
    ]
    if mode == 2:
        scratch += [
            pltpu.VMEM((NP,), jnp.float32),
            pltpu.VMEM((CHUNK,), jnp.float32),
            pltpu.VMEM((CHUNK,), jnp.float32),
            pltpu.VMEM((16,), jnp.float32),
        ]

    def body(*refs):
        it = iter(refs)
        h_hbm = next(it); asv_hbm = next(it); adv_hbm = next(it)
        src_hbm = next(it); dst_hbm = next(it)
        if mode == 2:
            ex2_hbm = next(it); s2t_hbm = next(it)
        outp_hbm = next(it); sp_hbm = next(it)
        if mode == 1:
            exh_hbm = next(it)
        if mode == 2:
            alh_hbm = next(it); cv_hbm = next(it)
        as_t = next(it); ad_t = next(it); rows_v = next(it)
        srcix = next(it); dstix = next(it); exf = next(it)
        out_sh = next(it); s_sh = next(it)
        gsa = next(it); gsb = next(it); ssa = next(it); ssb = next(it)
        if mode == 2:
            s2_t = next(it); ex2f = next(it); alf = next(it); cvb = next(it)

        cid = lax.axis_index("c")
        sid = lax.axis_index("s")
        wid = sid * NC + cid

        pltpu.sync_copy(asv_hbm, as_t)
        pltpu.sync_copy(adv_hbm, ad_t)

        zero16 = jnp.zeros((16,), jnp.float32)

        def mx_body(i, carry):
            ma, md = carry
            return (jnp.maximum(ma, as_t[pl.ds(i * 16, 16)]),
                    jnp.maximum(md, ad_t[pl.ds(i * 16, 16)]))
        ma, md = lax.fori_loop(0, NP // 16, mx_body,
                               (jnp.full((16,), NEG, jnp.float32),
                                jnp.full((16,), NEG, jnp.float32)))

        lanes = lax.iota(jnp.int32, 16)

        def allmax(v):
            for sh in (1, 2, 4, 8):
                exf[pl.ds(0, 16)] = v
                v = jnp.maximum(v, plsc.load_gather(exf, [lanes ^ sh]))
            return v

        C = allmax(ma) + allmax(md)

        if mode == 2:
            pltpu.sync_copy(s2t_hbm, s2_t)

        def zrow_body(r, _):
            for t in range(D // 16):
                rows_v[r, pl.ds(t * 16, 16)] = zero16
            return 0
        lax.fori_loop(0, CHUNK, zrow_body, 0)

        def zex_body(i, _):
            exf[pl.ds(i * 16, 16)] = zero16
            return 0
        lax.fori_loop(0, CHUNK // 16, zex_body, 0)

        base = sid * RPT
        for k in range(RPT // CHUNK):
            pltpu.sync_copy(rows_v, out_sh.at[pl.ds(base + k * CHUNK, CHUNK)])

        @pl.when(sid == 0)
        def _():
            for k in range(NP // CHUNK):
                pltpu.sync_copy(exf, s_sh.at[pl.ds(k * CHUNK, CHUNK)])

        plsc.subcore_barrier()

        nch_w = NCH // NW + jnp.where(wid < NCH % NW, 1, 0)

        def scale_half(base):
            def scale_body(r, _):
                for u in range(4):
                    rr = base + r * 4 + u
                    bc = plsc.load_gather(
                        exf, [jnp.broadcast_to(rr, (16,)).astype(jnp.int32)])
                    for t in range(D // 16):
                        sl = pl.ds(t * 16, 16)
                        rows_v[rr, sl] = rows_v[rr, sl] * bc
                return 0
            lax.fori_loop(0, HALF // 4, scale_body, 0)

        def chunk_body(k, _):
            ci = wid + k * NW
            pltpu.sync_copy(src_hbm.at[ci], srcix)
            pltpu.sync_copy(dst_hbm.at[ci], dstix)

            gA = pltpu.async_copy(h_hbm.at[srcix.at[0]],
                                  rows_v.at[pl.ds(0, HALF)], gsa)
            gB = pltpu.async_copy(h_hbm.at[srcix.at[1]],
                                  rows_v.at[pl.ds(HALF, HALF)], gsb)

            if mode == 2:
                pltpu.sync_copy(ex2_hbm.at[ci], ex2f)

            for j in range(2):
                for t in range(HALF // 16):
                    sl = pl.ds(j * HALF + t * 16, 16)
                    srcv = srcix[j, pl.ds(t * 16, 16)]
                    dstv = dstix[j, pl.ds(t * 16, 16)]
                    av = plsc.load_gather(as_t, [srcv])
                    dv = plsc.load_gather(ad_t, [dstv])
                    e = av + dv
                    e = jnp.maximum(e, 0.2 * e)
                    exf[sl] = jnp.exp(e - C)
                    if mode == 2:
                        s2g = plsc.load_gather(s2_t, [dstv])
                        alf[sl] = ex2f[sl] / (s2g + EPS)

            if mode == 1:
                pltpu.sync_copy(exf, exh_hbm.at[ci])
            if mode == 2:
                pltpu.sync_copy(alf, alh_hbm.at[ci])

            pltpu.sync_copy(exf.at[pl.ds(0, HALF)],
                            s_sh.at[dstix.at[0]], add=True)
            pltpu.sync_copy(exf.at[pl.ds(HALF, HALF)],
                            s_sh.at[dstix.at[1]], add=True)

            gA.wait()
            scale_half(0)
            dA = pltpu.async_copy(rows_v.at[pl.ds(0, HALF)],
                                  out_sh.at[dstix.at[0]], ssa, add=True)
            gB.wait()
            scale_half(HALF)
            dB = pltpu.async_copy(rows_v.at[pl.ds(HALF, HALF)],
                                  out_sh.at[dstix.at[1]], ssb, add=True)
            dA.wait()
            dB.wait()
            return 0

        lax.fori_loop(0, nch_w, chunk_body, 0)

        if mode == 2:
            @pl.when(wid == 0)
            def _():
                cvb[pl.ds(0, 16)] = C
                pltpu.sync_copy(cvb, cv_hbm)

        plsc.subcore_barrier()

        pltpu.sync_copy(out_sh.at[pl.ds(base, RPT)],
                        outp_hbm.at[cid, pl.ds(base, RPT)])
        pltpu.sync_copy(s_sh.at[pl.ds(base, RPT)],
                        sp_hbm.at[cid, pl.ds(base, RPT)])

    return pl.kernel(body, out_type=out_type, mesh=mesh,
                     scratch_types=scratch,
                     compiler_params=pltpu.CompilerParams(
                         needs_layout_passes=False))


_sc1 = _make_sc(0)
_sc2 = _make_sc(1)
_scv = _make_sc(2)


def kernel(x, edge_index, W1, a_src1, a_dst1, b1, W2, a_src2, a_dst2, b2,
           Wv, a_srcv, a_dstv, bv, lin_W, lin_b):
    xp = jnp.zeros((NP, D), jnp.float32).at[:N].set(x)
    src_r = edge_index[0].reshape(NCH, 2, HALF)
    dst_r = edge_index[1].reshape(NCH, 2, HALF)

    h1m, sa1, sd1 = _t1(xp, W1, a_src1, a_dst1)
    out1p, s1p = _sc1(h1m, sa1.reshape(NP), sd1.reshape(NP), src_r, dst_r)

    h2, hv, t2s, t2d, tvs, tvd = _t2(out1p, s1p, b1, W2, Wv,
                                     a_src2, a_dst2, a_srcv, a_dstv)
    out2p, s2p, exh = _sc2(h2, t2s.reshape(NP), t2d.reshape(NP), src_r, dst_r)
    s2tot = _t2b(s2p).reshape(NP)
    outvp, svp, alh, cv16 = _scv(hv, tvs.reshape(NP), tvd.reshape(NP),
                                 src_r, dst_r, exh, s2tot)

    mean, var = _t3(out2p, s2p, b2, outvp, svp, bv, tvs, tvd, hv,
                    jnp.tile(cv16, 8), lin_W, lin_b)
    return (mean[:N], var[:N], edge_index, alh.reshape(E))


# speedup vs baseline: 31.0232x; 1.0760x over previous
import functools

import jax
import jax.numpy as jnp
from jax import lax
from jax.experimental import pallas as pl
from jax.experimental.pallas import tpu as pltpu
from jax.experimental.pallas import tpu_sc as plsc

N = 10000
NP = 10240
D = 128
E = 320000
CHUNK = 128
HALF = CHUNK // 2
NCH = E // CHUNK
NC = 2
NT = 16
NW = NC * NT
RPT = NP // NT
TAB = 10048
BR = 1024
GRID = NP // BR
EPS = 1e-16
NEG = -3.0e38


def _t1_body(x_ref, w_ref, avs_ref, avd_ref, h_ref, sa_ref, sd_ref):
    h = jnp.dot(x_ref[...], w_ref[...], preferred_element_type=jnp.float32)
    h_ref[...] = h
    sa_ref[...] = jnp.sum(h * avs_ref[...][None, :], axis=1).reshape(8, 128)
    sd_ref[...] = jnp.sum(h * avd_ref[...][None, :], axis=1).reshape(8, 128)


def _t1(xp, W1, a_src1, a_dst1):
    return pl.pallas_call(
        _t1_body,
        grid=(GRID,),
        in_specs=[
            pl.BlockSpec((BR, D), lambda i: (i, 0)),
            pl.BlockSpec((D, D), lambda i: (0, 0)),
            pl.BlockSpec((D,), lambda i: (0,)),
            pl.BlockSpec((D,), lambda i: (0,)),
        ],
        out_specs=[
            pl.BlockSpec((BR, D), lambda i: (i, 0)),
            pl.BlockSpec((8, 128), lambda i: (i, 0)),
            pl.BlockSpec((8, 128), lambda i: (i, 0)),
        ],
        out_shape=[
            jax.ShapeDtypeStruct((NP, D), jnp.float32),
            jax.ShapeDtypeStruct((NP // 128, 128), jnp.float32),
            jax.ShapeDtypeStruct((NP // 128, 128), jnp.float32),
        ],
    )(xp, W1, a_src1, a_dst1)


def _t2_body(p_ref, s_ref, b1_ref, w2_ref, wv_ref, a2s_ref, a2d_ref,
             avs_ref, avd_ref,
             h2_ref, hv_ref, t2s_ref, t2d_ref, tvs_ref, tvd_ref):
    ssum = jnp.sum(s_ref[...], axis=0)
    p = p_ref[0] + p_ref[1]
    v = p * (1.0 / (ssum + EPS))[:, None] + b1_ref[...][None, :]
    h1 = jnp.where(v > 0, v, jnp.exp(v) - 1.0)
    h2 = jnp.dot(h1, w2_ref[...], preferred_element_type=jnp.float32)
    hv = jnp.dot(h1, wv_ref[...], preferred_element_type=jnp.float32)
    h2_ref[...] = h2
    hv_ref[...] = hv
    t2s_ref[...] = jnp.sum(h2 * a2s_ref[...][None, :], axis=1).reshape(8, 128)
    t2d_ref[...] = jnp.sum(h2 * a2d_ref[...][None, :], axis=1).reshape(8, 128)
    tvs_ref[...] = jnp.sum(hv * avs_ref[...][None, :], axis=1).reshape(8, 128)
    tvd_ref[...] = jnp.sum(hv * avd_ref[...][None, :], axis=1).reshape(8, 128)


def _t2(out1p, s1p, b1, W2, Wv, a_src2, a_dst2, a_srcv, a_dstv):
    vec = pl.BlockSpec((D,), lambda i: (0,))
    mat = pl.BlockSpec((D, D), lambda i: (0, 0))
    tab = pl.BlockSpec((8, 128), lambda i: (i, 0))
    return pl.pallas_call(
        _t2_body,
        grid=(GRID,),
        in_specs=[
            pl.BlockSpec((NC, BR, D), lambda i: (0, i, 0)),
            pl.BlockSpec((NC, BR), lambda i: (0, i)),
            vec, mat, mat, vec, vec, vec, vec,
        ],
        out_specs=[
            pl.BlockSpec((BR, D), lambda i: (i, 0)),
            pl.BlockSpec((BR, D), lambda i: (i, 0)),
            tab, tab, tab, tab,
        ],
        out_shape=[
            jax.ShapeDtypeStruct((NP, D), jnp.float32),
            jax.ShapeDtypeStruct((NP, D), jnp.float32),
            jax.ShapeDtypeStruct((NP // 128, 128), jnp.float32),
            jax.ShapeDtypeStruct((NP // 128, 128), jnp.float32),
            jax.ShapeDtypeStruct((NP // 128, 128), jnp.float32),
            jax.ShapeDtypeStruct((NP // 128, 128), jnp.float32),
        ],
    )(out1p, s1p, b1, W2, Wv, a_src2, a_dst2, a_srcv, a_dstv)


def _t2b_body(s_ref, o_ref):
    o_ref[...] = jnp.sum(s_ref[...], axis=0).reshape(8, 128)


def _t2b(s2p):
    return pl.pallas_call(
        _t2b_body,
        grid=(GRID,),
        in_specs=[pl.BlockSpec((NC, BR), lambda i: (0, i))],
        out_specs=pl.BlockSpec((8, 128), lambda i: (i, 0)),
        out_shape=jax.ShapeDtypeStruct((NP // 128, 128), jnp.float32),
    )(s2p)


def _t3_body(p2_ref, s2_ref, b2_ref, pv_ref, sv_ref, bv_ref,
             tvs_ref, tvd_ref, hv_ref, cv_ref, lw_ref, lb_ref,
             mean_ref, var_ref):
    s2 = jnp.sum(s2_ref[...], axis=0)
    m = (p2_ref[0] + p2_ref[1]) * (1.0 / (s2 + EPS))[:, None] + b2_ref[...][None, :]
    m = jnp.maximum(m, 0.0)
    mean_ref[...] = (
        jnp.dot(m, lw_ref[...], preferred_element_type=jnp.float32)
        + lb_ref[...][None, :]
    )
    cv = cv_ref[0]
    es = tvs_ref[...].reshape(BR) + tvd_ref[...].reshape(BR)
    es = jnp.maximum(es, 0.2 * es)
    exs = jnp.exp(es - cv)
    sv = jnp.sum(sv_ref[...], axis=0) + exs
    num = pv_ref[0] + pv_ref[1] + exs[:, None] * hv_ref[...]
    var_ref[...] = num * (1.0 / (sv + EPS))[:, None] + bv_ref[...][None, :]


def _t3(out2p, s2p, b2, outvp, svp, bv, tvs, tvd, hv, cv128, lin_W, lin_b):
    vec = pl.BlockSpec((D,), lambda i: (0,))
    mat = pl.BlockSpec((D, D), lambda i: (0, 0))
    tab = pl.BlockSpec((8, 128), lambda i: (i, 0))
    return pl.pallas_call(
        _t3_body,
        grid=(GRID,),
        in_specs=[
            pl.BlockSpec((NC, BR, D), lambda i: (0, i, 0)),
            pl.BlockSpec((NC, BR), lambda i: (0, i)),
            vec,
            pl.BlockSpec((NC, BR, D), lambda i: (0, i, 0)),
            pl.BlockSpec((NC, BR), lambda i: (0, i)),
            vec,
            tab, tab,
            pl.BlockSpec((BR, D), lambda i: (i, 0)),
            vec, mat, vec,
        ],
        out_specs=[
            pl.BlockSpec((BR, D), lambda i: (i, 0)),
            pl.BlockSpec((BR, D), lambda i: (i, 0)),
        ],
        out_shape=[
            jax.ShapeDtypeStruct((NP, D), jnp.float32),
            jax.ShapeDtypeStruct((NP, D), jnp.float32),
        ],
    )(out2p, s2p, b2, outvp, svp, bv, tvs, tvd, hv, cv128, lin_W, lin_b)


def _make_sc(mode):
    mesh = plsc.VectorSubcoreMesh(core_axis_name="c", subcore_axis_name="s")

    out_type = [
        jax.ShapeDtypeStruct((NC, NP, D), jnp.float32),
        jax.ShapeDtypeStruct((NC, NP), jnp.float32),
    ]
    if mode == 1:
        out_type.append(jax.ShapeDtypeStruct((NCH, CHUNK), jnp.float32))
    if mode == 2:
        out_type.append(jax.ShapeDtypeStruct((NCH, CHUNK), jnp.float32))
        out_type.append(jax.ShapeDtypeStruct((16,), jnp.float32))

    scratch = [
        pltpu.VMEM((TAB,), jnp.float32),
        pltpu.VMEM((TAB,), jnp.float32),
        pltpu.VMEM((CHUNK, D), jnp.float32),
        pltpu.VMEM((2, HALF), jnp.int32),
        pltpu.VMEM((2, HALF), jnp.int32),
        pltpu.VMEM((2, HALF), jnp.int32),
        pltpu.VMEM((2, HALF), jnp.int32),
        pltpu.VMEM((CHUNK,), jnp.float32),
        pltpu.VMEM_SHARED((NP, D), jnp.float32),
        pltpu.VMEM_SHARED((NP,), jnp.float32),
        pltpu.SemaphoreType.DMA,
        pltpu.SemaphoreType.DMA,
        pltpu.SemaphoreType.DMA,
        pltpu.SemaphoreType.DMA,
    ]
    if mode == 2:
        scratch += [
            pltpu.VMEM((TAB,), jnp.float32),
            pltpu.VMEM((CHUNK,), jnp.float32),
            pltpu.VMEM((CHUNK,), jnp.float32),
            pltpu.VMEM((16,), jnp.float32),
        ]

    def body(*refs):
        it = iter(refs)
        h_hbm = next(it); asv_hbm = next(it); adv_hbm = next(it)
        src_hbm = next(it); dst_hbm = next(it)
        if mode == 2:
            ex2_hbm = next(it); s2t_hbm = next(it)
        outp_hbm = next(it); sp_hbm = next(it)
        if mode == 1:
            exh_hbm = next(it)
        if mode == 2:
            alh_hbm = next(it); cv_hbm = next(it)
        as_t = next(it); ad_t = next(it); rows_v = next(it)
        srcix0 = next(it); dstix0 = next(it)
        srcix1 = next(it); dstix1 = next(it); exf = next(it)
        out_sh = next(it); s_sh = next(it)
        gsa = next(it); gsb = next(it); ssa = next(it); ssb = next(it)
        if mode == 2:
            s2_t = next(it); ex2f = next(it); alf = next(it); cvb = next(it)

        cid = lax.axis_index("c")
        sid = lax.axis_index("s")
        wid = sid * NC + cid

        pltpu.sync_copy(asv_hbm.at[pl.ds(0, TAB)], as_t)
        pltpu.sync_copy(adv_hbm.at[pl.ds(0, TAB)], ad_t)

        zero16 = jnp.zeros((16,), jnp.float32)

        def mx_body(i, carry):
            ma, md = carry
            return (jnp.maximum(ma, as_t[pl.ds(i * 16, 16)]),
                    jnp.maximum(md, ad_t[pl.ds(i * 16, 16)]))
        ma, md = lax.fori_loop(0, TAB // 16, mx_body,
                               (jnp.full((16,), NEG, jnp.float32),
                                jnp.full((16,), NEG, jnp.float32)))

        lanes = lax.iota(jnp.int32, 16)

        def allmax(v):
            for sh in (1, 2, 4, 8):
                exf[pl.ds(0, 16)] = v
                v = jnp.maximum(v, plsc.load_gather(exf, [lanes ^ sh]))
            return v

        C = allmax(ma) + allmax(md)

        if mode == 2:
            pltpu.sync_copy(s2t_hbm.at[pl.ds(0, TAB)], s2_t)

        def zrow_body(r, _):
            for t in range(D // 16):
                rows_v[r, pl.ds(t * 16, 16)] = zero16
            return 0
        lax.fori_loop(0, CHUNK, zrow_body, 0)

        def zex_body(i, _):
            exf[pl.ds(i * 16, 16)] = zero16
            return 0
        lax.fori_loop(0, CHUNK // 16, zex_body, 0)

        base = sid * RPT
        for k in range(RPT // CHUNK):
            pltpu.sync_copy(rows_v, out_sh.at[pl.ds(base + k * CHUNK, CHUNK)])

        @pl.when(sid == 0)
        def _():
            for k in range(NP // CHUNK):
                pltpu.sync_copy(exf, s_sh.at[pl.ds(k * CHUNK, CHUNK)])

        plsc.subcore_barrier()

        nch_w = NCH // NW + jnp.where(wid < NCH % NW, 1, 0)

        def scale_half(base):
            def scale_body(r, _):
                for u in range(4):
                    rr = base + r * 4 + u
                    bc = plsc.load_gather(
                        exf, [jnp.broadcast_to(rr, (16,)).astype(jnp.int32)])
                    for t in range(D // 16):
                        sl = pl.ds(t * 16, 16)
                        rows_v[rr, sl] = rows_v[rr, sl] * bc
                return 0
            lax.fori_loop(0, HALF // 4, scale_body, 0)

        def drain_rows():
            pltpu.make_async_copy(rows_v.at[pl.ds(0, HALF)],
                                  out_sh.at[dstix0.at[0]], ssa).wait()
            pltpu.make_async_copy(rows_v.at[pl.ds(HALF, HALF)],
                                  out_sh.at[dstix0.at[1]], ssb).wait()

        def chunk_step(ci, sx, dx, drain):
            pltpu.sync_copy(src_hbm.at[ci], sx)
            pltpu.sync_copy(dst_hbm.at[ci], dx)

            if drain is True:
                drain_rows()
            elif drain is not None:
                @pl.when(drain)
                def _():
                    drain_rows()

            gA = pltpu.async_copy(h_hbm.at[sx.at[0]],
                                  rows_v.at[pl.ds(0, HALF)], gsa)
            gB = pltpu.async_copy(h_hbm.at[sx.at[1]],
                                  rows_v.at[pl.ds(HALF, HALF)], gsb)

            if mode == 2:
                pltpu.sync_copy(ex2_hbm.at[ci], ex2f)

            for j in range(2):
                for t in range(HALF // 16):
                    sl = pl.ds(j * HALF + t * 16, 16)
                    srcv = sx[j, pl.ds(t * 16, 16)]
                    dstv = dx[j, pl.ds(t * 16, 16)]
                    av = plsc.load_gather(as_t, [srcv])
                    dv = plsc.load_gather(ad_t, [dstv])
                    e = av + dv
                    e = jnp.maximum(e, 0.2 * e)
                    exf[sl] = jnp.exp(e - C)
                    if mode == 2:
                        s2g = plsc.load_gather(s2_t, [dstv])
                        alf[sl] = ex2f[sl] / (s2g + EPS)

            if mode == 1:
                pltpu.sync_copy(exf, exh_hbm.at[ci])
            if mode == 2:
                pltpu.sync_copy(alf, alh_hbm.at[ci])

            pltpu.sync_copy(exf.at[pl.ds(0, HALF)],
                            s_sh.at[dx.at[0]], add=True)
            pltpu.sync_copy(exf.at[pl.ds(HALF, HALF)],
                            s_sh.at[dx.at[1]], add=True)

            gA.wait()
            scale_half(0)
            pltpu.async_copy(rows_v.at[pl.ds(0, HALF)],
                             out_sh.at[dx.at[0]], ssa, add=True)
            gB.wait()
            scale_half(HALF)
            pltpu.async_copy(rows_v.at[pl.ds(HALF, HALF)],
                             out_sh.at[dx.at[1]], ssb, add=True)

        def pair_body(p, _):
            k0 = 2 * p
            chunk_step(wid + k0 * NW, srcix0, dstix0, p > 0)
            chunk_step(wid + (k0 + 1) * NW, srcix1, dstix1, True)
            return 0

        lax.fori_loop(0, nch_w // 2, pair_body, 0)

        @pl.when(nch_w % 2 == 1)
        def _():
            chunk_step(wid + (nch_w - 1) * NW, srcix0, dstix0, True)

        drain_rows()

        if mode == 2:
            @pl.when(wid == 0)
            def _():
                cvb[pl.ds(0, 16)] = C
                pltpu.sync_copy(cvb, cv_hbm)

        plsc.subcore_barrier()

        pltpu.sync_copy(out_sh.at[pl.ds(base, RPT)],
                        outp_hbm.at[cid, pl.ds(base, RPT)])
        pltpu.sync_copy(s_sh.at[pl.ds(base, RPT)],
                        sp_hbm.at[cid, pl.ds(base, RPT)])

    return pl.kernel(body, out_type=out_type, mesh=mesh,
                     scratch_types=scratch,
                     compiler_params=pltpu.CompilerParams(
                         needs_layout_passes=False))


_sc1 = _make_sc(0)
_sc2 = _make_sc(1)
_scv = _make_sc(2)


def kernel(x, edge_index, W1, a_src1, a_dst1, b1, W2, a_src2, a_dst2, b2,
           Wv, a_srcv, a_dstv, bv, lin_W, lin_b):
    xp = jnp.zeros((NP, D), jnp.float32).at[:N].set(x)
    src_r = edge_index[0].reshape(NCH, 2, HALF)
    dst_r = edge_index[1].reshape(NCH, 2, HALF)

    h1m, sa1, sd1 = _t1(xp, W1, a_src1, a_dst1)
    out1p, s1p = _sc1(h1m, sa1.reshape(NP), sd1.reshape(NP), src_r, dst_r)

    h2, hv, t2s, t2d, tvs, tvd = _t2(out1p, s1p, b1, W2, Wv,
                                     a_src2, a_dst2, a_srcv, a_dstv)
    out2p, s2p, exh = _sc2(h2, t2s.reshape(NP), t2d.reshape(NP), src_r, dst_r)
    s2tot = _t2b(s2p).reshape(NP)
    outvp, svp, alh, cv16 = _scv(hv, tvs.reshape(NP), tvd.reshape(NP),
                                 src_r, dst_r, exh, s2tot)

    mean, var = _t3(out2p, s2p, b2, outvp, svp, bv, tvs, tvd, hv,
                    jnp.tile(cv16, 8), lin_W, lin_b)
    return (mean[:N], var[:N], edge_index, alh.reshape(E))


# speedup vs baseline: 31.8941x; 1.0281x over previous
import functools

import jax
import jax.numpy as jnp
from jax import lax
from jax.experimental import pallas as pl
from jax.experimental.pallas import tpu as pltpu
from jax.experimental.pallas import tpu_sc as plsc

N = 10000
NP = 10240
D = 128
E = 320000
CHUNK = 128
HALF = CHUNK // 2
NCH = E // CHUNK
NC = 2
NT = 16
NW = NC * NT
RPT = NP // NT
TAB = 10048
BR = 1024
GRID = NP // BR
EPS = 1e-16
NEG = -3.0e38


def _t1_body(x_ref, w_ref, avs_ref, avd_ref, h_ref, sa_ref, sd_ref):
    h = jnp.dot(x_ref[...], w_ref[...], preferred_element_type=jnp.float32)
    h_ref[...] = h
    sa_ref[...] = jnp.sum(h * avs_ref[...][None, :], axis=1).reshape(8, 128)
    sd_ref[...] = jnp.sum(h * avd_ref[...][None, :], axis=1).reshape(8, 128)


def _t1(xp, W1, a_src1, a_dst1):
    return pl.pallas_call(
        _t1_body,
        grid=(GRID,),
        in_specs=[
            pl.BlockSpec((BR, D), lambda i: (i, 0)),
            pl.BlockSpec((D, D), lambda i: (0, 0)),
            pl.BlockSpec((D,), lambda i: (0,)),
            pl.BlockSpec((D,), lambda i: (0,)),
        ],
        out_specs=[
            pl.BlockSpec((BR, D), lambda i: (i, 0)),
            pl.BlockSpec((8, 128), lambda i: (i, 0)),
            pl.BlockSpec((8, 128), lambda i: (i, 0)),
        ],
        out_shape=[
            jax.ShapeDtypeStruct((NP, D), jnp.float32),
            jax.ShapeDtypeStruct((NP // 128, 128), jnp.float32),
            jax.ShapeDtypeStruct((NP // 128, 128), jnp.float32),
        ],
    )(xp, W1, a_src1, a_dst1)


def _t2_body(p_ref, s_ref, b1_ref, w2_ref, wv_ref, a2s_ref, a2d_ref,
             avs_ref, avd_ref,
             h2_ref, hv_ref, t2s_ref, t2d_ref, tvs_ref, tvd_ref):
    ssum = jnp.sum(s_ref[...], axis=0)
    p = p_ref[0] + p_ref[1]
    v = p * (1.0 / (ssum + EPS))[:, None] + b1_ref[...][None, :]
    h1 = jnp.where(v > 0, v, jnp.exp(v) - 1.0)
    h2 = jnp.dot(h1, w2_ref[...], preferred_element_type=jnp.float32)
    hv = jnp.dot(h1, wv_ref[...], preferred_element_type=jnp.float32)
    h2_ref[...] = h2
    hv_ref[...] = hv
    t2s_ref[...] = jnp.sum(h2 * a2s_ref[...][None, :], axis=1).reshape(8, 128)
    t2d_ref[...] = jnp.sum(h2 * a2d_ref[...][None, :], axis=1).reshape(8, 128)
    tvs_ref[...] = jnp.sum(hv * avs_ref[...][None, :], axis=1).reshape(8, 128)
    tvd_ref[...] = jnp.sum(hv * avd_ref[...][None, :], axis=1).reshape(8, 128)


def _t2(out1p, s1p, b1, W2, Wv, a_src2, a_dst2, a_srcv, a_dstv):
    vec = pl.BlockSpec((D,), lambda i: (0,))
    mat = pl.BlockSpec((D, D), lambda i: (0, 0))
    tab = pl.BlockSpec((8, 128), lambda i: (i, 0))
    return pl.pallas_call(
        _t2_body,
        grid=(GRID,),
        in_specs=[
            pl.BlockSpec((NC, BR, D), lambda i: (0, i, 0)),
            pl.BlockSpec((NC, BR), lambda i: (0, i)),
            vec, mat, mat, vec, vec, vec, vec,
        ],
        out_specs=[
            pl.BlockSpec((BR, D), lambda i: (i, 0)),
            pl.BlockSpec((BR, D), lambda i: (i, 0)),
            tab, tab, tab, tab,
        ],
        out_shape=[
            jax.ShapeDtypeStruct((NP, D), jnp.float32),
            jax.ShapeDtypeStruct((NP, D), jnp.float32),
            jax.ShapeDtypeStruct((NP // 128, 128), jnp.float32),
            jax.ShapeDtypeStruct((NP // 128, 128), jnp.float32),
            jax.ShapeDtypeStruct((NP // 128, 128), jnp.float32),
            jax.ShapeDtypeStruct((NP // 128, 128), jnp.float32),
        ],
    )(out1p, s1p, b1, W2, Wv, a_src2, a_dst2, a_srcv, a_dstv)


def _t2b_body(s_ref, o_ref):
    o_ref[...] = jnp.sum(s_ref[...], axis=0).reshape(8, 128)


def _t2b(s2p):
    return pl.pallas_call(
        _t2b_body,
        grid=(GRID,),
        in_specs=[pl.BlockSpec((NC, BR), lambda i: (0, i))],
        out_specs=pl.BlockSpec((8, 128), lambda i: (i, 0)),
        out_shape=jax.ShapeDtypeStruct((NP // 128, 128), jnp.float32),
    )(s2p)


def _t3_body(p2_ref, s2_ref, b2_ref, pv_ref, sv_ref, bv_ref,
             tvs_ref, tvd_ref, hv_ref, cv_ref, lw_ref, lb_ref,
             mean_ref, var_ref):
    s2 = jnp.sum(s2_ref[...], axis=0)
    m = (p2_ref[0] + p2_ref[1]) * (1.0 / (s2 + EPS))[:, None] + b2_ref[...][None, :]
    m = jnp.maximum(m, 0.0)
    mean_ref[...] = (
        jnp.dot(m, lw_ref[...], preferred_element_type=jnp.float32)
        + lb_ref[...][None, :]
    )
    cv = cv_ref[0]
    es = tvs_ref[...].reshape(BR) + tvd_ref[...].reshape(BR)
    es = jnp.maximum(es, 0.2 * es)
    exs = jnp.exp(es - cv)
    sv = jnp.sum(sv_ref[...], axis=0) + exs
    num = pv_ref[0] + pv_ref[1] + exs[:, None] * hv_ref[...]
    var_ref[...] = num * (1.0 / (sv + EPS))[:, None] + bv_ref[...][None, :]


def _t3(out2p, s2p, b2, outvp, svp, bv, tvs, tvd, hv, cv128, lin_W, lin_b):
    vec = pl.BlockSpec((D,), lambda i: (0,))
    mat = pl.BlockSpec((D, D), lambda i: (0, 0))
    tab = pl.BlockSpec((8, 128), lambda i: (i, 0))
    return pl.pallas_call(
        _t3_body,
        grid=(GRID,),
        in_specs=[
            pl.BlockSpec((NC, BR, D), lambda i: (0, i, 0)),
            pl.BlockSpec((NC, BR), lambda i: (0, i)),
            vec,
            pl.BlockSpec((NC, BR, D), lambda i: (0, i, 0)),
            pl.BlockSpec((NC, BR), lambda i: (0, i)),
            vec,
            tab, tab,
            pl.BlockSpec((BR, D), lambda i: (i, 0)),
            vec, mat, vec,
        ],
        out_specs=[
            pl.BlockSpec((BR, D), lambda i: (i, 0)),
            pl.BlockSpec((BR, D), lambda i: (i, 0)),
        ],
        out_shape=[
            jax.ShapeDtypeStruct((NP, D), jnp.float32),
            jax.ShapeDtypeStruct((NP, D), jnp.float32),
        ],
    )(out2p, s2p, b2, outvp, svp, bv, tvs, tvd, hv, cv128, lin_W, lin_b)


def _make_sc(mode):
    mesh = plsc.VectorSubcoreMesh(core_axis_name="c", subcore_axis_name="s")

    out_type = [
        jax.ShapeDtypeStruct((NC, NP, D), jnp.float32),
        jax.ShapeDtypeStruct((NC, NP), jnp.float32),
    ]
    if mode == 1:
        out_type.append(jax.ShapeDtypeStruct((NCH, CHUNK), jnp.float32))
    if mode == 2:
        out_type.append(jax.ShapeDtypeStruct((NCH, CHUNK), jnp.float32))
        out_type.append(jax.ShapeDtypeStruct((16,), jnp.float32))

    scratch = [
        pltpu.VMEM((TAB,), jnp.float32),
        pltpu.VMEM((TAB,), jnp.float32),
        pltpu.VMEM((CHUNK, D), jnp.float32),
        pltpu.VMEM((4, HALF), jnp.int32),
        pltpu.VMEM((4, HALF), jnp.int32),
        pltpu.VMEM((CHUNK,), jnp.float32),
        pltpu.VMEM_SHARED((NP, D), jnp.float32),
        pltpu.VMEM_SHARED((NP,), jnp.float32),
        pltpu.SemaphoreType.DMA,
        pltpu.SemaphoreType.DMA,
        pltpu.SemaphoreType.DMA,
        pltpu.SemaphoreType.DMA,
    ]
    if mode == 2:
        scratch += [
            pltpu.VMEM((TAB,), jnp.float32),
            pltpu.VMEM((CHUNK,), jnp.float32),
            pltpu.VMEM((CHUNK,), jnp.float32),
            pltpu.VMEM((16,), jnp.float32),
        ]

    def body(*refs):
        it = iter(refs)
        h_hbm = next(it); asv_hbm = next(it); adv_hbm = next(it)
        edg_hbm = next(it)
        if mode == 2:
            ex2_hbm = next(it); s2t_hbm = next(it)
        outp_hbm = next(it); sp_hbm = next(it)
        if mode == 1:
            exh_hbm = next(it)
        if mode == 2:
            alh_hbm = next(it); cv_hbm = next(it)
        as_t = next(it); ad_t = next(it); rows_v = next(it)
        ix0 = next(it); ix1 = next(it); exf = next(it)
        out_sh = next(it); s_sh = next(it)
        gsa = next(it); gsb = next(it); ssa = next(it); ssb = next(it)
        if mode == 2:
            s2_t = next(it); ex2f = next(it); alf = next(it); cvb = next(it)

        cid = lax.axis_index("c")
        sid = lax.axis_index("s")
        wid = sid * NC + cid

        pltpu.sync_copy(asv_hbm.at[pl.ds(0, TAB)], as_t)
        pltpu.sync_copy(adv_hbm.at[pl.ds(0, TAB)], ad_t)

        zero16 = jnp.zeros((16,), jnp.float32)

        def mx_body(i, carry):
            ma, md = carry
            return (jnp.maximum(ma, as_t[pl.ds(i * 16, 16)]),
                    jnp.maximum(md, ad_t[pl.ds(i * 16, 16)]))
        ma, md = lax.fori_loop(0, TAB // 16, mx_body,
                               (jnp.full((16,), NEG, jnp.float32),
                                jnp.full((16,), NEG, jnp.float32)))

        lanes = lax.iota(jnp.int32, 16)

        def allmax(v):
            for sh in (1, 2, 4, 8):
                exf[pl.ds(0, 16)] = v
                v = jnp.maximum(v, plsc.load_gather(exf, [lanes ^ sh]))
            return v

        C = allmax(ma) + allmax(md)

        if mode == 2:
            pltpu.sync_copy(s2t_hbm.at[pl.ds(0, TAB)], s2_t)

        def zrow_body(r, _):
            for t in range(D // 16):
                rows_v[r, pl.ds(t * 16, 16)] = zero16
            return 0
        lax.fori_loop(0, CHUNK, zrow_body, 0)

        def zex_body(i, _):
            exf[pl.ds(i * 16, 16)] = zero16
            return 0
        lax.fori_loop(0, CHUNK // 16, zex_body, 0)

        base = sid * RPT
        for k in range(RPT // CHUNK):
            pltpu.sync_copy(rows_v, out_sh.at[pl.ds(base + k * CHUNK, CHUNK)])

        @pl.when(sid == 0)
        def _():
            for k in range(NP // CHUNK):
                pltpu.sync_copy(exf, s_sh.at[pl.ds(k * CHUNK, CHUNK)])

        plsc.subcore_barrier()

        nch_w = NCH // NW + jnp.where(wid < NCH % NW, 1, 0)

        def scale_half(base):
            def scale_body(r, _):
                for u in range(4):
                    rr = base + r * 4 + u
                    bc = plsc.load_gather(
                        exf, [jnp.broadcast_to(rr, (16,)).astype(jnp.int32)])
                    for t in range(D // 16):
                        sl = pl.ds(t * 16, 16)
                        rows_v[rr, sl] = rows_v[rr, sl] * bc
                return 0
            lax.fori_loop(0, HALF // 4, scale_body, 0)

        def drain_rows():
            pltpu.make_async_copy(rows_v.at[pl.ds(0, HALF)],
                                  out_sh.at[ix0.at[2]], ssa).wait()
            pltpu.make_async_copy(rows_v.at[pl.ds(HALF, HALF)],
                                  out_sh.at[ix0.at[3]], ssb).wait()

        def chunk_step(ci, ix, drain):

            if drain is True:
                drain_rows()
            elif drain is not None:
                @pl.when(drain)
                def _():
                    drain_rows()

            pltpu.sync_copy(edg_hbm.at[ci], ix)

            gA = pltpu.async_copy(h_hbm.at[ix.at[0]],
                                  rows_v.at[pl.ds(0, HALF)], gsa)
            gB = pltpu.async_copy(h_hbm.at[ix.at[1]],
                                  rows_v.at[pl.ds(HALF, HALF)], gsb)

            if mode == 2:
                pltpu.sync_copy(ex2_hbm.at[ci], ex2f)

            for j in range(2):
                for t in range(HALF // 16):
                    sl = pl.ds(j * HALF + t * 16, 16)
                    srcv = ix[j, pl.ds(t * 16, 16)]
                    dstv = ix[2 + j, pl.ds(t * 16, 16)]
                    av = plsc.load_gather(as_t, [srcv])
                    dv = plsc.load_gather(ad_t, [dstv])
                    e = av + dv
                    e = jnp.maximum(e, 0.2 * e)
                    exf[sl] = jnp.exp(e - C)
                    if mode == 2:
                        s2g = plsc.load_gather(s2_t, [dstv])
                        alf[sl] = ex2f[sl] / (s2g + EPS)

            if mode == 1:
                pltpu.sync_copy(exf, exh_hbm.at[ci])
            if mode == 2:
                pltpu.sync_copy(alf, alh_hbm.at[ci])

            pltpu.sync_copy(exf.at[pl.ds(0, HALF)],
                            s_sh.at[ix.at[2]], add=True)
            pltpu.sync_copy(exf.at[pl.ds(HALF, HALF)],
                            s_sh.at[ix.at[3]], add=True)

            gA.wait()
            scale_half(0)
            pltpu.async_copy(rows_v.at[pl.ds(0, HALF)],
                             out_sh.at[ix.at[2]], ssa, add=True)
            gB.wait()
            scale_half(HALF)
            pltpu.async_copy(rows_v.at[pl.ds(HALF, HALF)],
                             out_sh.at[ix.at[3]], ssb, add=True)

        def pair_body(p, _):
            k0 = 2 * p
            chunk_step(wid + k0 * NW, ix0, p > 0)
            chunk_step(wid + (k0 + 1) * NW, ix1, True)
            return 0

        lax.fori_loop(0, nch_w // 2, pair_body, 0)

        @pl.when(nch_w % 2 == 1)
        def _():
            chunk_step(wid + (nch_w - 1) * NW, ix0, True)

        drain_rows()

        if mode == 2:
            @pl.when(wid == 0)
            def _():
                cvb[pl.ds(0, 16)] = C
                pltpu.sync_copy(cvb, cv_hbm)

        plsc.subcore_barrier()

        pltpu.sync_copy(out_sh.at[pl.ds(base, RPT)],
                        outp_hbm.at[cid, pl.ds(base, RPT)])
        pltpu.sync_copy(s_sh.at[pl.ds(base, RPT)],
                        sp_hbm.at[cid, pl.ds(base, RPT)])

    return pl.kernel(body, out_type=out_type, mesh=mesh,
                     scratch_types=scratch,
                     compiler_params=pltpu.CompilerParams(
                         needs_layout_passes=False))


_sc1 = _make_sc(0)
_sc2 = _make_sc(1)
_scv = _make_sc(2)


def kernel(x, edge_index, W1, a_src1, a_dst1, b1, W2, a_src2, a_dst2, b2,
           Wv, a_srcv, a_dstv, bv, lin_W, lin_b):
    xp = jnp.zeros((NP, D), jnp.float32).at[:N].set(x)
    edges_p = jnp.concatenate([edge_index[0].reshape(NCH, 2, HALF),
                               edge_index[1].reshape(NCH, 2, HALF)], axis=1)

    h1m, sa1, sd1 = _t1(xp, W1, a_src1, a_dst1)
    out1p, s1p = _sc1(h1m, sa1.reshape(NP), sd1.reshape(NP), edges_p)

    h2, hv, t2s, t2d, tvs, tvd = _t2(out1p, s1p, b1, W2, Wv,
                                     a_src2, a_dst2, a_srcv, a_dstv)
    out2p, s2p, exh = _sc2(h2, t2s.reshape(NP), t2d.reshape(NP), edges_p)
    s2tot = _t2b(s2p).reshape(NP)
    outvp, svp, alh, cv16 = _scv(hv, tvs.reshape(NP), tvd.reshape(NP),
                                 edges_p, exh, s2tot)

    mean, var = _t3(out2p, s2p, b2, outvp, svp, bv, tvs, tvd, hv,
                    jnp.tile(cv16, 8), lin_W, lin_b)
    return (mean[:N], var[:N], edge_index, alh.reshape(E))
